# Initial kernel scaffold; baseline (speedup 1.0000x reference)
import jax
import jax.numpy as jnp
from jax.experimental import pallas as pl


def kernel(x_att_discrete, tables):
    raise NotImplementedError("write your pallas kernel here")


# speedup vs baseline: 2.4093x; 2.4093x over previous
import functools

import jax
import jax.numpy as jnp
from jax import lax
from jax.experimental import pallas as pl
from jax.experimental.pallas import tpu as pltpu
from jax.experimental.pallas import tpu_sc as plsc

B = 16384
NUM_FIELDS = 26
VOCAB = 100
DIM = 99

NC, NS, L = 2, 16, 16
NW = NC * NS
TOTAL = B * NUM_FIELDS
PER_TILE = TOTAL // NW
CHUNK = 512
GATHER_W = 128
NG = CHUNK // GATHER_W
N_CHUNKS = PER_TILE // CHUNK


def _body(x_hbm, tab_hbm, out_hbm, xv, idxv, rows, sem, osem):
    wid = lax.axis_index("s") * NC + lax.axis_index("c")
    tile_base = wid * PER_TILE

    @pl.loop(0, N_CHUNKS)
    def _chunk(s):
        base = tile_base + s * CHUNK
        pltpu.sync_copy(x_hbm.at[pl.ds(base, CHUNK)], xv)
        iota = lax.iota(jnp.int32, L)
        for g in range(NG):
            for m in range(GATHER_W // L):
                off = g * GATHER_W + m * L
                pos = base + off + iota
                field = lax.rem(pos, NUM_FIELDS)
                idxv[g, pl.ds(m * L, L)] = xv[pl.ds(off, L)] + field * VOCAB
        copies = []
        for g in range(NG):
            copies.append(
                pltpu.async_copy(tab_hbm.at[idxv.at[g]], rows.at[g], sem))
        for c in copies:
            c.wait()
        pltpu.async_copy(rows, out_hbm.at[pl.ds(base // GATHER_W, NG)],
                         osem).wait()


@functools.partial(jax.jit, static_argnames=())
def kernel(x_att_discrete, tables):
    x_flat = x_att_discrete.astype(jnp.int32).reshape(TOTAL)
    tab = tables.reshape(NUM_FIELDS * VOCAB, DIM)
    run = pl.kernel(
        _body,
        out_type=jax.ShapeDtypeStruct((TOTAL // GATHER_W, GATHER_W, DIM),
                                      jnp.float32),
        mesh=plsc.VectorSubcoreMesh(core_axis_name="c", subcore_axis_name="s"),
        scratch_types=[
            pltpu.VMEM((CHUNK,), jnp.int32),
            pltpu.VMEM((NG, GATHER_W), jnp.int32),
            pltpu.VMEM((NG, GATHER_W, DIM), jnp.float32),
            pltpu.SemaphoreType.DMA,
            pltpu.SemaphoreType.DMA,
        ],
        compiler_params=pltpu.CompilerParams(use_tc_tiling_on_sc=False),
    )
    out = run(x_flat, tab)
    return out.reshape(B, NUM_FIELDS * DIM)


# speedup vs baseline: 2.6107x; 1.0836x over previous
import jax
import jax.numpy as jnp
from jax import lax
from jax.experimental import pallas as pl
from jax.experimental.pallas import tpu as pltpu
from jax.experimental.pallas import tpu_sc as plsc

B = 16384
NUM_FIELDS = 26
VOCAB = 100
DIM = 99
DIM_PAD = 112

NC, NS, L = 2, 16, 16
NW = NC * NS
TOTAL = B * NUM_FIELDS
PER_TILE = TOTAL // NW
CHUNK = 256
GATHER_W = 128
NG = CHUNK // GATHER_W
N_CHUNKS = PER_TILE // CHUNK
NVEC = DIM_PAD // L


def _body(x_hbm, tab_hbm, out_hbm, xv, idxv, rows, packed, gsem, osem):
    wid = lax.axis_index("s") * NC + lax.axis_index("c")
    tile_base = wid * PER_TILE

    @pl.loop(0, N_CHUNKS)
    def _chunk(s):
        base = tile_base + s * CHUNK
        pltpu.sync_copy(x_hbm.at[pl.ds(base, CHUNK)], xv)
        iota = lax.iota(jnp.int32, L)
        for g in range(NG):
            for m in range(GATHER_W // L):
                off = g * GATHER_W + m * L
                pos = base + off + iota
                field = lax.rem(pos, NUM_FIELDS)
                idxv[g, pl.ds(m * L, L)] = xv[pl.ds(off, L)] + field * VOCAB
        copies = [
            pltpu.async_copy(tab_hbm.at[idxv.at[g]], rows.at[g], gsem)
            for g in range(NG)
        ]
        for c in copies:
            c.wait()
        for r in range(CHUNK):
            for j in range(NVEC):
                packed[pl.ds(r * DIM + L * j, L)] = \
                    rows[r // GATHER_W, r % GATHER_W, pl.ds(L * j, L)]
        pltpu.async_copy(packed.at[pl.ds(0, CHUNK * DIM)],
                         out_hbm.at[pl.ds(base * DIM, CHUNK * DIM)],
                         osem).wait()


def kernel(x_att_discrete, tables):
    x_flat = x_att_discrete.astype(jnp.int32).reshape(TOTAL)
    tab = jnp.pad(tables.reshape(NUM_FIELDS * VOCAB, DIM),
                  ((0, 0), (0, DIM_PAD - DIM)))
    run = pl.kernel(
        _body,
        out_type=jax.ShapeDtypeStruct((TOTAL * DIM,), jnp.float32),
        mesh=plsc.VectorSubcoreMesh(core_axis_name="c", subcore_axis_name="s"),
        scratch_types=[
            pltpu.VMEM((CHUNK,), jnp.int32),
            pltpu.VMEM((NG, GATHER_W), jnp.int32),
            pltpu.VMEM((NG, GATHER_W, DIM_PAD), jnp.float32),
            pltpu.VMEM((CHUNK * DIM + L,), jnp.float32),
            pltpu.SemaphoreType.DMA,
            pltpu.SemaphoreType.DMA,
        ],
        compiler_params=pltpu.CompilerParams(use_tc_tiling_on_sc=False),
    )
    out = run(x_flat, tab)
    return out.reshape(B, NUM_FIELDS * DIM)


# speedup vs baseline: 3.4574x; 1.3243x over previous
import jax
import jax.numpy as jnp
from jax import lax
from jax.experimental import pallas as pl
from jax.experimental.pallas import tpu as pltpu
from jax.experimental.pallas import tpu_sc as plsc

B = 16384
NUM_FIELDS = 26
VOCAB = 100
DIM = 99
DIM_PAD = 112

NC, NS, L = 2, 16, 16
NW = NC * NS
TOTAL = B * NUM_FIELDS
PER_TILE = TOTAL // NW
GATHER_W = 128
CHUNK = 128
NG = CHUNK // GATHER_W
N_CHUNKS = PER_TILE // CHUNK
NVEC = DIM_PAD // L
IDX_ROWS = PER_TILE // GATHER_W
STAGE_ROWS = (NUM_FIELDS * VOCAB + NS - 1) // NS
TAB_ROWS = STAGE_ROWS * NS
PACK_LEN = CHUNK * DIM


def _body(x_hbm, tab_hbm, out_hbm, shared_tab, idxall, rows, packed,
          gsem0, gsem1, osem0, osem1):
    cid = lax.axis_index("c")
    sid = lax.axis_index("s")
    wid = sid * NC + cid
    tile_base = wid * PER_TILE

    r0 = sid * STAGE_ROWS
    pltpu.sync_copy(tab_hbm.at[pl.ds(r0, STAGE_ROWS)],
                    shared_tab.at[pl.ds(r0, STAGE_ROWS)])
    plsc.subcore_barrier()

    pltpu.sync_copy(x_hbm.at[pl.ds(wid * IDX_ROWS, IDX_ROWS)], idxall)
    iota = lax.iota(jnp.int32, L)

    @pl.loop(0, IDX_ROWS)
    def _idx(r):
        row_base = (wid * IDX_ROWS + r) * GATHER_W
        for m in range(GATHER_W // L):
            pos = row_base + m * L + iota
            field = lax.rem(pos, NUM_FIELDS)
            idxall[r, pl.ds(m * L, L)] = \
                idxall[r, pl.ds(m * L, L)] + field * VOCAB

    gsems = (gsem0, gsem1)
    osems = (osem0, osem1)

    def fire_gathers(c, p):
        for g in range(NG):
            pltpu.async_copy(
                shared_tab.at[idxall.at[c * NG + g]],
                rows.at[p, pl.ds(g * GATHER_W, GATHER_W)],
                gsems[p])

    def drain_gathers(c, p):
        for g in range(NG):
            pltpu.make_async_copy(
                shared_tab.at[idxall.at[c * NG + g]],
                rows.at[p, pl.ds(g * GATHER_W, GATHER_W)],
                gsems[p]).wait()

    def drain_out(c, p):
        pltpu.make_async_copy(
            packed.at[p, pl.ds(0, PACK_LEN)],
            out_hbm.at[pl.ds((tile_base + c * CHUNK) * DIM, PACK_LEN)],
            osems[p]).wait()

    fire_gathers(0, 0)

    @pl.loop(0, N_CHUNKS, step=2)
    def _chunk(s):
        for c_off in range(2):
            p = c_off
            c = s + c_off
            nxt = c + 1

            @pl.when(nxt < N_CHUNKS)
            def _():
                fire_gathers(nxt, 1 - p)
            drain_gathers(c, p)

            @pl.when(c >= 2)
            def _():
                drain_out(c - 2, p)
            for r in range(CHUNK):
                for j in range(NVEC):
                    packed[p, pl.ds(r * DIM + L * j, L)] = \
                        rows[p, r, pl.ds(L * j, L)]
            pltpu.async_copy(
                packed.at[p, pl.ds(0, PACK_LEN)],
                out_hbm.at[pl.ds((tile_base + c * CHUNK) * DIM, PACK_LEN)],
                osems[p])

    drain_out(N_CHUNKS - 2, 0)
    drain_out(N_CHUNKS - 1, 1)


def kernel(x_att_discrete, tables):
    x2d = x_att_discrete.astype(jnp.int32).reshape(TOTAL // GATHER_W, GATHER_W)
    tab = jnp.pad(tables.reshape(NUM_FIELDS * VOCAB, DIM),
                  ((0, TAB_ROWS - NUM_FIELDS * VOCAB), (0, DIM_PAD - DIM)))
    run = pl.kernel(
        _body,
        out_type=jax.ShapeDtypeStruct((TOTAL * DIM,), jnp.float32),
        mesh=plsc.VectorSubcoreMesh(core_axis_name="c", subcore_axis_name="s"),
        scratch_types=[
            pltpu.VMEM_SHARED((TAB_ROWS, DIM_PAD), jnp.float32),
            pltpu.VMEM((IDX_ROWS, GATHER_W), jnp.int32),
            pltpu.VMEM((2, CHUNK, DIM_PAD), jnp.float32),
            pltpu.VMEM((2, PACK_LEN + L), jnp.float32),
            pltpu.SemaphoreType.DMA,
            pltpu.SemaphoreType.DMA,
            pltpu.SemaphoreType.DMA,
            pltpu.SemaphoreType.DMA,
        ],
        compiler_params=pltpu.CompilerParams(use_tc_tiling_on_sc=False),
    )
    out = run(x2d, tab)
    return out.reshape(B, NUM_FIELDS * DIM)


# speedup vs baseline: 3.9265x; 1.1357x over previous
import jax
import jax.numpy as jnp
from jax import lax
from jax.experimental import pallas as pl
from jax.experimental.pallas import tpu as pltpu
from jax.experimental.pallas import tpu_sc as plsc

B = 16384
NUM_FIELDS = 26
VOCAB = 100
DIM = 99
DIM_PAD = 112
OUTW = NUM_FIELDS * DIM
PADW = 2688

NC, NS, L = 2, 16, 16
NW = NC * NS
TOTAL = B * NUM_FIELDS
PER_TILE = TOTAL // NW
ROWS_C = 4
CHUNK = ROWS_C * NUM_FIELDS
N_CHUNKS = PER_TILE // CHUNK
NVEC = DIM_PAD // L
IDX_VECS = PER_TILE // L
STAGE_ROWS = (NUM_FIELDS * VOCAB + NS - 1) // NS
TAB_ROWS = STAGE_ROWS * NS
PACK_LEN = ROWS_C * PADW

ROWS_B = 64


def _sc_body(x_hbm, tab_hbm, out_hbm, shared_tab, idxall, rows, packed,
             gsem0, gsem1, osem0, osem1):
    cid = lax.axis_index("c")
    sid = lax.axis_index("s")
    wid = sid * NC + cid
    tile_base = wid * PER_TILE

    r0 = sid * STAGE_ROWS
    pltpu.sync_copy(tab_hbm.at[pl.ds(r0, STAGE_ROWS)],
                    shared_tab.at[pl.ds(r0, STAGE_ROWS)])
    plsc.subcore_barrier()

    pltpu.sync_copy(x_hbm.at[pl.ds(tile_base, PER_TILE)], idxall)
    iota = lax.iota(jnp.int32, L)

    @pl.loop(0, IDX_VECS)
    def _idx(r):
        pos = tile_base + r * L + iota
        field = lax.rem(pos, NUM_FIELDS)
        idxall[pl.ds(r * L, L)] = idxall[pl.ds(r * L, L)] + field * VOCAB

    gsems = (gsem0, gsem1)
    osems = (osem0, osem1)

    def gather_descr(c, p):
        return pltpu.make_async_copy(
            shared_tab.at[idxall.at[pl.ds(c * CHUNK, CHUNK)]],
            rows.at[p], gsems[p])

    def out_descr(c, p):
        return pltpu.make_async_copy(
            packed.at[p, pl.ds(0, PACK_LEN)],
            out_hbm.at[pl.ds((wid * N_CHUNKS + c) * PACK_LEN, PACK_LEN)],
            osems[p])

    gather_descr(0, 0).start()

    @pl.loop(0, N_CHUNKS, step=2)
    def _chunk(s):
        for c_off in range(2):
            p = c_off
            c = s + c_off
            nxt = c + 1

            @pl.when(nxt < N_CHUNKS)
            def _():
                gather_descr(nxt, 1 - p).start()
            gather_descr(c, p).wait()

            @pl.when(c >= 2)
            def _():
                out_descr(c - 2, p).wait()
            for r in range(ROWS_C):
                for i in range(NUM_FIELDS):
                    for j in range(NVEC):
                        packed[p, pl.ds(r * PADW + i * DIM + L * j, L)] = \
                            rows[p, r * NUM_FIELDS + i, pl.ds(L * j, L)]
            out_descr(c, p).start()

    out_descr(N_CHUNKS - 2, 0).wait()
    out_descr(N_CHUNKS - 1, 1).wait()


def _relayout_body(in_ref, out_ref):
    for r in range(ROWS_B):
        out_ref[r, :] = in_ref[pl.ds(r * PADW, OUTW)]


def kernel(x_att_discrete, tables):
    x_flat = x_att_discrete.astype(jnp.int32).reshape(TOTAL)
    tab = jnp.pad(tables.reshape(NUM_FIELDS * VOCAB, DIM),
                  ((0, TAB_ROWS - NUM_FIELDS * VOCAB), (0, DIM_PAD - DIM)))
    gather = pl.kernel(
        _sc_body,
        out_type=jax.ShapeDtypeStruct((B * PADW,), jnp.float32),
        mesh=plsc.VectorSubcoreMesh(core_axis_name="c", subcore_axis_name="s"),
        scratch_types=[
            pltpu.VMEM_SHARED((TAB_ROWS, DIM_PAD), jnp.float32),
            pltpu.VMEM((PER_TILE,), jnp.int32),
            pltpu.VMEM((2, CHUNK, DIM_PAD), jnp.float32),
            pltpu.VMEM((2, PACK_LEN + L), jnp.float32),
            pltpu.SemaphoreType.DMA,
            pltpu.SemaphoreType.DMA,
            pltpu.SemaphoreType.DMA,
            pltpu.SemaphoreType.DMA,
        ],
        compiler_params=pltpu.CompilerParams(use_tc_tiling_on_sc=False),
    )
    flat = gather(x_flat, tab)
    out = pl.pallas_call(
        _relayout_body,
        grid=(B // ROWS_B,),
        in_specs=[pl.BlockSpec((ROWS_B * PADW,), lambda m: (m,))],
        out_specs=pl.BlockSpec((ROWS_B, OUTW), lambda m: (m, 0)),
        out_shape=jax.ShapeDtypeStruct((B, OUTW), jnp.float32),
    )(flat)
    return out


# speedup vs baseline: 4.5744x; 1.1650x over previous
import jax
import jax.numpy as jnp
from jax import lax
from jax.experimental import pallas as pl
from jax.experimental.pallas import tpu as pltpu
from jax.experimental.pallas import tpu_sc as plsc

B = 16384
NUM_FIELDS = 26
VOCAB = 100
DIM = 99
DIM_PAD = 112
OUTW = NUM_FIELDS * DIM
PADW = 2688

NC, NS, L = 2, 16, 16
NW = NC * NS
TOTAL = B * NUM_FIELDS
PER_TILE = TOTAL // NW
ROWS_C = 4
CHUNK = ROWS_C * NUM_FIELDS
N_CHUNKS = PER_TILE // CHUNK
NVEC = DIM_PAD // L
IDX_VECS = PER_TILE // L
STAGE_ROWS = (NUM_FIELDS * VOCAB + NS - 1) // NS
TAB_ROWS = STAGE_ROWS * NS
PACK_LEN = ROWS_C * PADW

ROWS_B = 64


def _sc_body(x_hbm, tab_hbm, out_hbm, shared_tab, idxall, rows, packed,
             gsem0, gsem1, osem0, osem1):
    cid = lax.axis_index("c")
    sid = lax.axis_index("s")
    wid = sid * NC + cid
    tile_base = wid * PER_TILE

    r0 = sid * STAGE_ROWS
    pltpu.sync_copy(tab_hbm.at[pl.ds(r0, STAGE_ROWS)],
                    shared_tab.at[pl.ds(r0, STAGE_ROWS)])
    plsc.subcore_barrier()

    pltpu.sync_copy(x_hbm.at[pl.ds(tile_base, PER_TILE)], idxall)
    iota = lax.iota(jnp.int32, L)

    @pl.loop(0, IDX_VECS)
    def _idx(r):
        pos = tile_base + r * L + iota
        field = lax.rem(pos, NUM_FIELDS)
        idxall[pl.ds(r * L, L)] = idxall[pl.ds(r * L, L)] + field * VOCAB

    gsems = (gsem0, gsem1)
    osems = (osem0, osem1)

    def gather_descr(c, p):
        return pltpu.make_async_copy(
            shared_tab.at[idxall.at[pl.ds(c * CHUNK, CHUNK)]],
            rows.at[p], gsems[p])

    def out_descr(c, p):
        return pltpu.make_async_copy(
            packed.at[p, pl.ds(0, PACK_LEN)],
            out_hbm.at[pl.ds((wid * N_CHUNKS + c) * PACK_LEN, PACK_LEN)],
            osems[p])

    gather_descr(0, 0).start()

    @pl.loop(0, N_CHUNKS, step=2)
    def _chunk(s):
        for c_off in range(2):
            p = c_off
            c = s + c_off
            nxt = c + 1

            @pl.when(nxt < N_CHUNKS)
            def _():
                gather_descr(nxt, 1 - p).start()
            gather_descr(c, p).wait()

            @pl.when(c >= 2)
            def _():
                out_descr(c - 2, p).wait()
            for r in range(ROWS_C):
                for i in range(NUM_FIELDS):
                    for j in range(NVEC):
                        packed[p, pl.ds(r * PADW + i * DIM + L * j, L)] = \
                            rows[p, r * NUM_FIELDS + i, pl.ds(L * j, L)]
            out_descr(c, p).start()

    out_descr(N_CHUNKS - 2, 0).wait()
    out_descr(N_CHUNKS - 1, 1).wait()


def _relayout_body(in_ref, out_ref):
    for r in range(ROWS_B):
        out_ref[r, :] = in_ref[pl.ds(r * PADW, OUTW)]


def kernel(x_att_discrete, tables):
    x_flat = x_att_discrete.astype(jnp.int32).reshape(TOTAL)
    tab = jnp.pad(tables.reshape(NUM_FIELDS * VOCAB, DIM),
                  ((0, TAB_ROWS - NUM_FIELDS * VOCAB), (0, DIM_PAD - DIM)))
    gather = pl.kernel(
        _sc_body,
        out_type=jax.ShapeDtypeStruct((B * PADW,), jnp.float32),
        mesh=plsc.VectorSubcoreMesh(core_axis_name="c", subcore_axis_name="s"),
        scratch_types=[
            pltpu.VMEM_SHARED((TAB_ROWS, DIM_PAD), jnp.float32),
            pltpu.VMEM((PER_TILE,), jnp.int32),
            pltpu.VMEM((2, CHUNK, DIM_PAD), jnp.float32),
            pltpu.VMEM((2, PACK_LEN + L), jnp.float32),
            pltpu.SemaphoreType.DMA,
            pltpu.SemaphoreType.DMA,
            pltpu.SemaphoreType.DMA,
            pltpu.SemaphoreType.DMA,
        ],
        compiler_params=pltpu.CompilerParams(use_tc_tiling_on_sc=False),
    )
    flat = gather(x_flat, tab)
    return flat.reshape(B, PADW)[:, :OUTW]


# speedup vs baseline: 4.5782x; 1.0008x over previous
import jax
import jax.numpy as jnp
from jax import lax
from jax.experimental import pallas as pl
from jax.experimental.pallas import tpu as pltpu
from jax.experimental.pallas import tpu_sc as plsc

B = 16384
NUM_FIELDS = 26
VOCAB = 100
DIM = 99
DIM_PAD = 112
OUTW = NUM_FIELDS * DIM
PADW = 2688

NC, NS, L = 2, 16, 16
NW = NC * NS
TOTAL = B * NUM_FIELDS
PER_TILE = TOTAL // NW
ROWS_C = 4
CHUNK = ROWS_C * NUM_FIELDS
N_CHUNKS = PER_TILE // CHUNK
NVEC = DIM_PAD // L
IDX_VECS = PER_TILE // L
STAGE_ROWS = (NUM_FIELDS * VOCAB + NS - 1) // NS
TAB_ROWS = STAGE_ROWS * NS
PACK_LEN = ROWS_C * PADW

def _sc_body(x_hbm, tab_hbm, out_hbm, shared_tab, idxall, rows, packed,
             gsem0, gsem1, osem0, osem1):
    cid = lax.axis_index("c")
    sid = lax.axis_index("s")
    wid = sid * NC + cid
    tile_base = wid * PER_TILE

    r0 = sid * STAGE_ROWS
    pltpu.sync_copy(tab_hbm.at[pl.ds(r0, STAGE_ROWS)],
                    shared_tab.at[pl.ds(r0, STAGE_ROWS)])
    plsc.subcore_barrier()

    pltpu.sync_copy(x_hbm.at[pl.ds(tile_base, PER_TILE)], idxall)
    iota = lax.iota(jnp.int32, L)

    @pl.loop(0, IDX_VECS)
    def _idx(r):
        pos = tile_base + r * L + iota
        field = lax.rem(pos, NUM_FIELDS)
        idxall[pl.ds(r * L, L)] = idxall[pl.ds(r * L, L)] + field * VOCAB

    gsems = (gsem0, gsem1)
    osems = (osem0, osem1)

    def gather_descr(c, p):
        return pltpu.make_async_copy(
            shared_tab.at[idxall.at[pl.ds(c * CHUNK, CHUNK)]],
            rows.at[p], gsems[p])

    def out_descr(c, p):
        return pltpu.make_async_copy(
            packed.at[p],
            out_hbm.at[pl.ds((wid * N_CHUNKS + c) * ROWS_C, ROWS_C)],
            osems[p])

    gather_descr(0, 0).start()

    @pl.loop(0, N_CHUNKS, step=2)
    def _chunk(s):
        for c_off in range(2):
            p = c_off
            c = s + c_off
            nxt = c + 1

            @pl.when(nxt < N_CHUNKS)
            def _():
                gather_descr(nxt, 1 - p).start()
            gather_descr(c, p).wait()

            @pl.when(c >= 2)
            def _():
                out_descr(c - 2, p).wait()
            for r in range(ROWS_C):
                for i in range(NUM_FIELDS):
                    for j in range(NVEC):
                        packed[p, r, pl.ds(i * DIM + L * j, L)] = \
                            rows[p, r * NUM_FIELDS + i, pl.ds(L * j, L)]
            out_descr(c, p).start()

    out_descr(N_CHUNKS - 2, 0).wait()
    out_descr(N_CHUNKS - 1, 1).wait()


def kernel(x_att_discrete, tables):
    x_flat = x_att_discrete.astype(jnp.int32).reshape(TOTAL)
    tab = jnp.pad(tables.reshape(NUM_FIELDS * VOCAB, DIM),
                  ((0, TAB_ROWS - NUM_FIELDS * VOCAB), (0, DIM_PAD - DIM)))
    gather = pl.kernel(
        _sc_body,
        out_type=jax.ShapeDtypeStruct((B, PADW), jnp.float32),
        mesh=plsc.VectorSubcoreMesh(core_axis_name="c", subcore_axis_name="s"),
        scratch_types=[
            pltpu.VMEM_SHARED((TAB_ROWS, DIM_PAD), jnp.float32),
            pltpu.VMEM((PER_TILE,), jnp.int32),
            pltpu.VMEM((2, CHUNK, DIM_PAD), jnp.float32),
            pltpu.VMEM((2, ROWS_C, PADW), jnp.float32),
            pltpu.SemaphoreType.DMA,
            pltpu.SemaphoreType.DMA,
            pltpu.SemaphoreType.DMA,
            pltpu.SemaphoreType.DMA,
        ],
        compiler_params=pltpu.CompilerParams(use_tc_tiling_on_sc=False),
    )
    padded = gather(x_flat, tab)
    return padded[:, :OUTW]
